# baseline (speedup 1.0000x reference)
import functools

import jax
import jax.numpy as jnp
from jax import lax
from jax.experimental import pallas as pl
from jax.experimental.pallas import tpu as pltpu

_S = 128
_MAXSN = 64
_MAXSP = 128
_E = 768
_H = 128
_HE = 64
_NEG = -1e9
_F32 = jnp.float32


def _sig(x):
    return jax.nn.sigmoid(x)


def _enc_kernel(emb_ref, wid_ref, w0f_ref, w0b_ref, wih_ref, whh_ref, benc_ref,
                wenc_ref, mask_ref,
                pooled_ref, xihf_ref, xihb_ref, s0f_ref, s0b_ref, s1f_ref,
                s1b_ref):
    bb = emb_ref.shape[0]

    def pool_body(b, _):
        wrow = wid_ref[pl.ds(b, 1), :]
        iw = lax.broadcasted_iota(jnp.int32, (_MAXSN, _S), 0)
        oh = (iw == wrow).astype(_F32)
        pooled_ref[b] = jnp.dot(oh, emb_ref[b],
                                preferred_element_type=_F32)
        return 0

    lax.fori_loop(0, bb, pool_body, 0)
    mask_ref[...] = (jnp.sum(pooled_ref[...], axis=2) != 0.0).astype(_F32)

    def lstm_dir(xih_ref, whh_idx, out_ref, reverse):
        whh = whh_ref[whh_idx]

        def chunk(c, carry):
            base = pl.multiple_of(
                (_MAXSN - 8) - 8 * c if reverse else 8 * c, 8)
            xch = xih_ref[:, pl.ds(base, 8), :]
            h, cc = carry
            slots = [None] * 8
            order = range(7, -1, -1) if reverse else range(8)
            for j in order:
                g = xch[:, j, :] + jnp.dot(h, whh,
                                           preferred_element_type=_F32)
                i_ = g[:, 0:_HE]
                f_ = g[:, _HE:2 * _HE]
                g_ = g[:, 2 * _HE:3 * _HE]
                o_ = g[:, 3 * _HE:4 * _HE]
                cc = _sig(f_) * cc + _sig(i_) * jnp.tanh(g_)
                h = _sig(o_) * jnp.tanh(cc)
                slots[j] = h
            out_ref[:, pl.ds(base, 8), :] = jnp.concatenate(
                [s[:, None, :] for s in slots], axis=1)
            return (h, cc)

        init = (jnp.zeros((bb, _HE), _F32), jnp.zeros((bb, _HE), _F32))
        lax.fori_loop(0, _MAXSN // 8, chunk, init)

    pairs = [(s0f_ref, s0b_ref), (s1f_ref, s1b_ref)]

    for l in range(8):
        out_f, out_b = pairs[l % 2]
        if l == 0:
            pin = pooled_ref[...].reshape(bb * _MAXSN, _E)
            xihf_ref[...] = (jnp.dot(pin, w0f_ref[...],
                                     preferred_element_type=_F32)
                             + benc_ref[0:1, :]).reshape(bb, _MAXSN, 4 * _HE)
            xihb_ref[...] = (jnp.dot(pin, w0b_ref[...],
                                     preferred_element_type=_F32)
                             + benc_ref[1:2, :]).reshape(bb, _MAXSN, 4 * _HE)
        else:
            in_f, in_b = pairs[(l - 1) % 2]
            xf = in_f[...].reshape(bb * _MAXSN, _HE)
            xb = in_b[...].reshape(bb * _MAXSN, _HE)
            for di, xih_ref in ((0, xihf_ref), (1, xihb_ref)):
                w = wih_ref[2 * (l - 1) + di]
                xih_ref[...] = (jnp.dot(xf, w[0:_HE, :],
                                        preferred_element_type=_F32)
                                + jnp.dot(xb, w[_HE:2 * _HE, :],
                                          preferred_element_type=_F32)
                                + benc_ref[2 * l + di:2 * l + di + 1, :]
                                ).reshape(bb, _MAXSN, 4 * _HE)
        lstm_dir(xihf_ref, 2 * l, out_f, False)
        lstm_dir(xihb_ref, 2 * l + 1, out_b, True)
        if l >= 2:
            in_f, in_b = pairs[(l - 1) % 2]
            out_f[...] = out_f[...] + in_f[...]
            out_b[...] = out_b[...] + in_b[...]

    fin_f, fin_b = pairs[7 % 2]
    wenc_ref[:, :, 0:_HE] = fin_f[...]
    wenc_ref[:, :, _HE:2 * _HE] = fin_b[...]


def _prep_kernel(tok_ref, posg_ref, wid_ref, lng_ref, lnb_ref, w0_ref, b0_ref,
                 out_ref):
    bb = tok_ref.shape[0]

    def body(b, _):
        wrow = wid_ref[pl.ds(b, 1), :]
        iw = lax.broadcasted_iota(jnp.int32, (_MAXSP, _MAXSP), 0)
        oh = (iw == wrow).astype(_F32)
        x = jnp.dot(oh, tok_ref[b], preferred_element_type=_F32) \
            + posg_ref[b]
        m = jnp.mean(x, axis=-1, keepdims=True)
        xc = x - m
        v = jnp.mean(xc * xc, axis=-1, keepdims=True)
        xn = xc * lax.rsqrt(v + 1e-12) * lng_ref[...] + lnb_ref[...]
        out_ref[b] = jnp.dot(xn, w0_ref[...],
                             preferred_element_type=_F32) + b0_ref[...]
        return 0

    lax.fori_loop(0, bb, body, 0)


def _dec_kernel(xall_ref, wenc_ref, snlen_ref, mask_ref, pos_ref,
                wihd_ref, whhd_ref, bd_ref,
                wam_ref, wal_ref, ba_ref, bal_ref,
                wd1a_ref, wd1l_ref, wd1b_ref, b1_ref,
                wd2_ref, b2_ref, wd3_ref, b3_ref, wd4_ref, b4_ref,
                wd5_ref, b5_ref,
                out_ref):
    bb = xall_ref.shape[0]
    snlen = snlen_ref[...]
    bad0 = mask_ref[...] == 0.0
    iw = lax.broadcasted_iota(jnp.int32, (bb, _MAXSN), 1)

    def cell(g, c):
        i_ = g[:, 0:_H]
        f_ = g[:, _H:2 * _H]
        g_ = g[:, 2 * _H:3 * _H]
        o_ = g[:, 3 * _H:4 * _H]
        c2 = _sig(f_) * c + _sig(i_) * jnp.tanh(g_)
        h2 = _sig(o_) * jnp.tanh(c2)
        return h2, c2

    def chunk(cidx, carry):
        hs = list(carry[0:8])
        cs = list(carry[8:16])
        base = pl.multiple_of(8 * cidx, 8)
        xch = xall_ref[:, pl.ds(base, 8), :]
        pch = pos_ref[:, pl.ds(base, 8), :]
        outs = []
        for j in range(8):
            xt = xch[:, j, :]
            g0 = xt + jnp.dot(hs[0], whhd_ref[0],
                              preferred_element_type=_F32)
            hs[0], cs[0] = cell(g0, cs[0])
            cur = None
            for l in range(1, 8):
                src = hs[0] if l == 1 else cur
                gl = (jnp.dot(src, wihd_ref[l - 1],
                              preferred_element_type=_F32)
                      + jnp.dot(hs[l], whhd_ref[l],
                                preferred_element_type=_F32)
                      + bd_ref[l - 1:l, :])
                hs[l], cs[l] = cell(gl, cs[l])
                cur = hs[l] if l == 1 else hs[l] + cur
            in8 = cur
            a128 = jnp.dot(in8, wam_ref[...],
                           preferred_element_type=_F32) + ba_ref[...]
            alast = jnp.sum(in8 * wal_ref[...], axis=-1,
                            keepdims=True) + bal_ref[...]
            wenc = wenc_ref[...]
            prod = jnp.sum(a128[:, None, :] * wenc, axis=-1) \
                + alast * snlen
            pos_t = pch[:, j, :]
            left = jnp.maximum(pos_t - 1, 0)
            right = jnp.minimum(pos_t + 1, _MAXSN - 1)
            bad = bad0 | (iw < left) | (iw > right)
            prod = prod + jnp.where(bad, _NEG, 0.0)
            mx = jnp.max(prod, axis=-1, keepdims=True)
            ex = jnp.exp(prod - mx)
            wgt = ex / jnp.sum(ex, axis=-1, keepdims=True)
            ctx = jnp.sum(wgt[:, :, None] * wenc, axis=1)
            ctxl = jnp.sum(wgt * snlen, axis=-1, keepdims=True)
            z = jnp.maximum(
                jnp.dot(ctx, wd1a_ref[...], preferred_element_type=_F32)
                + ctxl * wd1l_ref[...]
                + jnp.dot(in8, wd1b_ref[...], preferred_element_type=_F32)
                + b1_ref[...], 0.0)
            z = jnp.maximum(jnp.dot(z, wd2_ref[...],
                                    preferred_element_type=_F32)
                            + b2_ref[...], 0.0)
            z = jnp.maximum(jnp.dot(z, wd3_ref[...],
                                    preferred_element_type=_F32)
                            + b3_ref[...], 0.0)
            z = jnp.maximum(jnp.dot(z, wd4_ref[...],
                                    preferred_element_type=_F32)
                            + b4_ref[...], 0.0)
            o = jnp.dot(z, wd5_ref[...],
                        preferred_element_type=_F32) + b5_ref[...]
            outs.append(o)
        out_ref[:, pl.ds(base, 8), :] = jnp.concatenate(
            [o[:, None, :] for o in outs], axis=1)
        return tuple(hs) + tuple(cs)

    zeros = jnp.zeros((bb, _H), _F32)
    init = tuple(zeros for _ in range(16))
    lax.fori_loop(0, _MAXSP // 8, chunk, init)


def _full(shape):
    nd = len(shape)
    return pl.BlockSpec(shape, lambda i: (0,) * nd)


def _blk(shape):
    nd = len(shape)
    return pl.BlockSpec(shape, lambda i, _nd=nd: (i,) + (0,) * (_nd - 1))


def _cparams(vmem_mb=56):
    return pltpu.CompilerParams(dimension_semantics=("parallel",),
                                vmem_limit_bytes=vmem_mb * 1024 * 1024)


@jax.jit
def kernel(sn_bert_emb, sn_word_len, params, word_ids_sn, sp_emd, sp_pos,
           word_ids_sp):
    b = sn_bert_emb.shape[0]
    f32 = _F32

    enc = params['enc']
    w0f = enc[0]['f']['Wih'].T
    w0b = enc[0]['b']['Wih'].T
    wih_enc = jnp.stack([enc[l][d]['Wih'].T
                         for l in range(1, 8) for d in ('f', 'b')])
    whh_enc = jnp.stack([enc[l][d]['Whh'].T
                         for l in range(8) for d in ('f', 'b')])
    benc = jnp.stack([enc[l][d]['bih'] + enc[l][d]['bhh']
                      for l in range(8) for d in ('f', 'b')])

    bb1 = min(32, b)
    wenc_x, maskf = pl.pallas_call(
        _enc_kernel,
        grid=(b // bb1,),
        in_specs=[
            _blk((bb1, _S, _E)),
            _blk((bb1, _S)),
            _full((_E, 4 * _HE)),
            _full((_E, 4 * _HE)),
            _full((14, 2 * _HE, 4 * _HE)),
            _full((16, _HE, 4 * _HE)),
            _full((16, 4 * _HE)),
        ],
        out_specs=[_blk((bb1, _MAXSN, _H)), _blk((bb1, _MAXSN))],
        out_shape=[jax.ShapeDtypeStruct((b, _MAXSN, _H), f32),
                   jax.ShapeDtypeStruct((b, _MAXSN), f32)],
        scratch_shapes=[
            pltpu.VMEM((bb1, _MAXSN, _E), f32),
            pltpu.VMEM((bb1, _MAXSN, 4 * _HE), f32),
            pltpu.VMEM((bb1, _MAXSN, 4 * _HE), f32),
            pltpu.VMEM((bb1, _MAXSN, _HE), f32),
            pltpu.VMEM((bb1, _MAXSN, _HE), f32),
            pltpu.VMEM((bb1, _MAXSN, _HE), f32),
            pltpu.VMEM((bb1, _MAXSN, _HE), f32),
        ],
        compiler_params=_cparams(),
        name="eyet_encode",
    )(sn_bert_emb, word_ids_sn, w0f, w0b, wih_enc, whh_enc, benc)

    tok = jnp.take(params['word_emb'], sp_emd[:, :-1], axis=0)
    posg = jnp.take(params['pos_emb'], sp_pos[:, :-1], axis=0)
    zrow = jnp.zeros((b, 1, _E), f32)
    tok_p = jnp.concatenate([tok, zrow], axis=1)
    posg_p = jnp.concatenate([posg, zrow], axis=1)
    wid_sp = jnp.concatenate(
        [word_ids_sp[:, :-1],
         jnp.full((b, 1), _MAXSP - 1, word_ids_sp.dtype)], axis=1)

    dec = params['dec']
    w0d = dec[0]['Wih'].T
    b0d = (dec[0]['bih'] + dec[0]['bhh'])[None, :]
    lng = params['ln_g'][None, :]
    lnb = params['ln_b'][None, :]

    bb2 = min(16, b)
    xall = pl.pallas_call(
        _prep_kernel,
        grid=(b // bb2,),
        in_specs=[
            _blk((bb2, _MAXSP, _E)),
            _blk((bb2, _MAXSP, _E)),
            _blk((bb2, _MAXSP)),
            _full((1, _E)),
            _full((1, _E)),
            _full((_E, 4 * _H)),
            _full((1, 4 * _H)),
        ],
        out_specs=_blk((bb2, _MAXSP, 4 * _H)),
        out_shape=jax.ShapeDtypeStruct((b, _MAXSP, 4 * _H), f32),
        compiler_params=_cparams(),
        name="eyet_prep",
    )(tok_p, posg_p, wid_sp, lng, lnb, w0d, b0d)

    wihd = jnp.stack([dec[l]['Wih'].T for l in range(1, 8)])
    whhd = jnp.stack([dec[l]['Whh'].T for l in range(8)])
    bd = jnp.stack([dec[l]['bih'] + dec[l]['bhh']
                    for l in range(1, 8)])
    bd = jnp.concatenate([bd, jnp.zeros((1, 4 * _H), f32)])

    wat = params['attn']['W'].T
    wam = wat[:, 0:_H]
    wal = wat[:, _H][None, :]
    ba = params['attn']['b'][None, 0:_H]
    bal = params['attn']['b'][None, _H:_H + 1]

    wd1t = params['d1']['W'].T
    wd1a = wd1t[0:_H, :]
    wd1l = wd1t[_H:_H + 1, :]
    wd1b = wd1t[_H + 1:2 * _H + 1, :]
    b1 = params['d1']['b'][None, :]
    wd2 = params['d2']['W'].T
    b2 = params['d2']['b'][None, :]
    wd3 = params['d3']['W'].T
    b3 = params['d3']['b'][None, :]
    wd4 = params['d4']['W'].T
    b4 = params['d4']['b'][None, :]
    wd5 = params['d5']['W'].T
    b5 = params['d5']['b'][None, :]
    nout = 2 * _MAXSN - 3

    pos3 = sp_pos[:, :, None].astype(jnp.int32)

    bb3 = min(32, b)
    out = pl.pallas_call(
        _dec_kernel,
        grid=(b // bb3,),
        in_specs=[
            _blk((bb3, _MAXSP, 4 * _H)),
            _blk((bb3, _MAXSN, _H)),
            _blk((bb3, _MAXSN)),
            _blk((bb3, _MAXSN)),
            _blk((bb3, _MAXSP, 1)),
            _full((7, _H, 4 * _H)),
            _full((8, _H, 4 * _H)),
            _full((8, 4 * _H)),
            _full((_H, _H)),
            _full((1, _H)),
            _full((1, _H)),
            _full((1, 1)),
            _full((_H, 512)),
            _full((1, 512)),
            _full((_H, 512)),
            _full((1, 512)),
            _full((512, 256)),
            _full((1, 256)),
            _full((256, 256)),
            _full((1, 256)),
            _full((256, 256)),
            _full((1, 256)),
            _full((256, nout)),
            _full((1, nout)),
        ],
        out_specs=_blk((bb3, _MAXSP, nout)),
        out_shape=jax.ShapeDtypeStruct((b, _MAXSP, nout), f32),
        compiler_params=_cparams(),
        name="eyet_decode",
    )(xall, wenc_x, sn_word_len, maskf, pos3,
      wihd, whhd, bd, wam, wal, ba, bal,
      wd1a, wd1l, wd1b, b1, wd2, b2, wd3, b3, wd4, b4, wd5, b5)

    return out[:, :_MAXSP - 1, :]


# speedup vs baseline: 1.1455x; 1.1455x over previous
import jax
import jax.numpy as jnp
from jax import lax
from jax.experimental import pallas as pl
from jax.experimental.pallas import tpu as pltpu

_S = 128
_MAXSN = 64
_MAXSP = 128
_E = 768
_H = 128
_HE = 64
_NEG = -1e9
_F32 = jnp.float32


def _sig(x):
    return jax.nn.sigmoid(x)


def _enc_kernel(emb_ref, wid_ref, w0f_ref, w0b_ref, wih_ref, whh_ref, benc_ref,
                wenc_ref, mask_ref,
                pooled_ref, xihf_ref, xihb_ref, xtf_ref, xtb_ref,
                s0f_ref, s0b_ref, s1f_ref, s1b_ref):
    bb = emb_ref.shape[0]

    def pool_body(b, _):
        wrow = wid_ref[pl.ds(b, 1), :]
        iw = lax.broadcasted_iota(jnp.int32, (_MAXSN, _S), 0)
        oh = (iw == wrow).astype(_F32)
        pooled_ref[b] = jnp.dot(oh, emb_ref[b],
                                preferred_element_type=_F32)
        return 0

    lax.fori_loop(0, bb, pool_body, 0)
    mask_ref[...] = (jnp.sum(pooled_ref[...], axis=2) != 0.0).astype(_F32)

    pin = pooled_ref[...].reshape(bb * _MAXSN, _E)
    xihf_ref[...] = (jnp.dot(pin, w0f_ref[...], preferred_element_type=_F32)
                     + benc_ref[0:1, :]).reshape(bb, _MAXSN, 4 * _HE)
    xihb_ref[...] = (jnp.dot(pin, w0b_ref[...], preferred_element_type=_F32)
                     + benc_ref[1:2, :]).reshape(bb, _MAXSN, 4 * _HE)

    def gates(g, h, c):
        i_ = g[:, 0:_HE]
        f_ = g[:, _HE:2 * _HE]
        g_ = g[:, 2 * _HE:3 * _HE]
        o_ = g[:, 3 * _HE:4 * _HE]
        c2 = _sig(f_) * c + _sig(i_) * jnp.tanh(g_)
        h2 = _sig(o_) * jnp.tanh(c2)
        return h2, c2

    def lstm_dir0(xih_ref, whh_idx, out_ref, reverse):
        whh = whh_ref[whh_idx]

        def chunk(c, carry):
            base = pl.multiple_of(
                (_MAXSN - 8) - 8 * c if reverse else 8 * c, 8)
            xch = xih_ref[:, pl.ds(base, 8), :]
            h, cc = carry
            order = range(7, -1, -1) if reverse else range(8)
            for j in order:
                g = xch[:, j, :] + jnp.dot(h, whh,
                                           preferred_element_type=_F32)
                h, cc = gates(g, h, cc)
                out_ref[base + j] = h
            return (h, cc)

        init = (jnp.zeros((bb, _HE), _F32), jnp.zeros((bb, _HE), _F32))
        lax.fori_loop(0, _MAXSN // 8, chunk, init)

    def lstm_dirT(xih_ref, whh_idx, out_ref, reverse):
        whh = whh_ref[whh_idx]

        def step(i, carry):
            t = (_MAXSN - 1) - i if reverse else i
            h, cc = carry
            g = xih_ref[t] + jnp.dot(h, whh,
                                        preferred_element_type=_F32)
            h, cc = gates(g, h, cc)
            out_ref[t] = h
            return (h, cc)

        init = (jnp.zeros((bb, _HE), _F32), jnp.zeros((bb, _HE), _F32))
        lax.fori_loop(0, _MAXSN, step, init)

    pairs = [(s0f_ref, s0b_ref), (s1f_ref, s1b_ref)]

    for l in range(8):
        out_f, out_b = pairs[l % 2]
        if l == 0:
            lstm_dir0(xihf_ref, 0, out_f, False)
            lstm_dir0(xihb_ref, 1, out_b, True)
        else:
            in_f, in_b = pairs[(l - 1) % 2]
            xf = in_f[...].reshape(_MAXSN * bb, _HE)
            xb = in_b[...].reshape(_MAXSN * bb, _HE)
            for di, xih_ref in ((0, xtf_ref), (1, xtb_ref)):
                w = wih_ref[2 * (l - 1) + di]
                xih_ref[...] = (jnp.dot(xf, w[0:_HE, :],
                                        preferred_element_type=_F32)
                                + jnp.dot(xb, w[_HE:2 * _HE, :],
                                          preferred_element_type=_F32)
                                + benc_ref[2 * l + di:2 * l + di + 1, :]
                                ).reshape(_MAXSN, bb, 4 * _HE)
            lstm_dirT(xtf_ref, 2 * l, out_f, False)
            lstm_dirT(xtb_ref, 2 * l + 1, out_b, True)
            if l >= 2:
                out_f[...] = out_f[...] + in_f[...]
                out_b[...] = out_b[...] + in_b[...]

    fin_f, fin_b = pairs[7 % 2]
    wenc_ref[:, :, 0:_HE] = fin_f[...]
    wenc_ref[:, :, _HE:2 * _HE] = fin_b[...]


def _prep_kernel(tok_ref, posg_ref, wid_ref, lng_ref, lnb_ref, w0_ref, b0_ref,
                 out_ref, xn_ref):
    bb = tok_ref.shape[0]

    def body(b, _):
        wrow = wid_ref[pl.ds(b, 1), :]
        iw = lax.broadcasted_iota(jnp.int32, (_MAXSP, _MAXSP), 0)
        oh = (iw == wrow).astype(_F32)
        x = jnp.dot(oh, tok_ref[b], preferred_element_type=_F32) \
            + posg_ref[b]
        m = jnp.mean(x, axis=-1, keepdims=True)
        xc = x - m
        v = jnp.mean(xc * xc, axis=-1, keepdims=True)
        xn_ref[b] = xc * lax.rsqrt(v + 1e-12) * lng_ref[...] + lnb_ref[...]
        return 0

    lax.fori_loop(0, bb, body, 0)
    out_ref[...] = (jnp.dot(xn_ref[...].reshape(bb * _MAXSP, _E), w0_ref[...],
                            preferred_element_type=_F32)
                    + b0_ref[...]).reshape(bb, _MAXSP, 4 * _H)


def _dec_kernel(xall_ref, wencT_ref, snlen_ref, negadd_ref,
                wihd_ref, whhd_ref, bd_ref,
                wam_ref, wal_ref, ba_ref, bal_ref,
                wd1a_ref, wd1l_ref, wd1b_ref, b1_ref,
                wd2_ref, b2_ref, wd3_ref, b3_ref, wd4_ref, b4_ref,
                wd5_ref, b5_ref,
                out_ref):
    bb = xall_ref.shape[1]
    snlen = snlen_ref[...]

    def cell(g, c):
        i_ = g[:, 0:_H]
        f_ = g[:, _H:2 * _H]
        g_ = g[:, 2 * _H:3 * _H]
        o_ = g[:, 3 * _H:4 * _H]
        c2 = _sig(f_) * c + _sig(i_) * jnp.tanh(g_)
        h2 = _sig(o_) * jnp.tanh(c2)
        return h2, c2

    def step(t, carry):
        hs = list(carry[0:8])
        cs = list(carry[8:16])
        xt = xall_ref[t]
        g0 = xt + jnp.dot(hs[0], whhd_ref[0], preferred_element_type=_F32)
        hs[0], cs[0] = cell(g0, cs[0])
        cur = None
        for l in range(1, 8):
            src = hs[0] if l == 1 else cur
            gl = (jnp.dot(src, wihd_ref[l - 1], preferred_element_type=_F32)
                  + jnp.dot(hs[l], whhd_ref[l], preferred_element_type=_F32)
                  + bd_ref[l - 1:l, :])
            hs[l], cs[l] = cell(gl, cs[l])
            cur = hs[l] if l == 1 else hs[l] + cur
        in8 = cur
        a128 = jnp.dot(in8, wam_ref[...],
                       preferred_element_type=_F32) + ba_ref[...]
        alast = jnp.sum(in8 * wal_ref[...], axis=-1,
                        keepdims=True) + bal_ref[...]
        cols = [jnp.sum(a128 * wencT_ref[w], axis=-1, keepdims=True)
                for w in range(_MAXSN)]
        prod = jnp.concatenate(cols, axis=1) \
            + alast * snlen + negadd_ref[t]
        mx = jnp.max(prod, axis=-1, keepdims=True)
        ex = jnp.exp(prod - mx)
        wgt = ex / jnp.sum(ex, axis=-1, keepdims=True)
        ctx = wgt[:, 0:1] * wencT_ref[0][0]
        for w in range(1, _MAXSN):
            ctx = ctx + wgt[:, w:w + 1] * wencT_ref[w]
        ctxl = jnp.sum(wgt * snlen, axis=-1, keepdims=True)
        z = jnp.maximum(
            jnp.dot(ctx, wd1a_ref[...], preferred_element_type=_F32)
            + ctxl * wd1l_ref[...]
            + jnp.dot(in8, wd1b_ref[...], preferred_element_type=_F32)
            + b1_ref[...], 0.0)
        z = jnp.maximum(jnp.dot(z, wd2_ref[...],
                                preferred_element_type=_F32) + b2_ref[...],
                        0.0)
        z = jnp.maximum(jnp.dot(z, wd3_ref[...],
                                preferred_element_type=_F32) + b3_ref[...],
                        0.0)
        z = jnp.maximum(jnp.dot(z, wd4_ref[...],
                                preferred_element_type=_F32) + b4_ref[...],
                        0.0)
        o = jnp.dot(z, wd5_ref[...],
                    preferred_element_type=_F32) + b5_ref[...]
        out_ref[t] = o
        return tuple(hs) + tuple(cs)

    zeros = jnp.zeros((bb, _H), _F32)
    lax.fori_loop(0, _MAXSP, step, tuple(zeros for _ in range(16)))


def _full(shape):
    nd = len(shape)
    return pl.BlockSpec(shape, lambda i: (0,) * nd)


def _blk(shape):
    nd = len(shape)
    return pl.BlockSpec(shape, lambda i, _nd=nd: (i,) + (0,) * (_nd - 1))


def _blk1(shape):
    nd = len(shape)
    return pl.BlockSpec(shape, lambda i, _nd=nd: (0, i) + (0,) * (_nd - 2))


def _cparams(vmem_mb=56):
    return pltpu.CompilerParams(dimension_semantics=("parallel",),
                                vmem_limit_bytes=vmem_mb * 1024 * 1024)


@jax.jit
def kernel(sn_bert_emb, sn_word_len, params, word_ids_sn, sp_emd, sp_pos,
           word_ids_sp):
    b = sn_bert_emb.shape[0]
    f32 = _F32

    enc = params['enc']
    w0f = enc[0]['f']['Wih'].T
    w0b = enc[0]['b']['Wih'].T
    wih_enc = jnp.stack([enc[l][d]['Wih'].T
                         for l in range(1, 8) for d in ('f', 'b')])
    whh_enc = jnp.stack([enc[l][d]['Whh'].T
                         for l in range(8) for d in ('f', 'b')])
    benc = jnp.stack([enc[l][d]['bih'] + enc[l][d]['bhh']
                      for l in range(8) for d in ('f', 'b')])

    bb1 = min(32, b)
    wencT, maskf = pl.pallas_call(
        _enc_kernel,
        grid=(b // bb1,),
        in_specs=[
            _blk((bb1, _S, _E)),
            _blk((bb1, _S)),
            _full((_E, 4 * _HE)),
            _full((_E, 4 * _HE)),
            _full((14, 2 * _HE, 4 * _HE)),
            _full((16, _HE, 4 * _HE)),
            _full((16, 4 * _HE)),
        ],
        out_specs=[_blk1((_MAXSN, bb1, _H)), _blk((bb1, _MAXSN))],
        out_shape=[jax.ShapeDtypeStruct((_MAXSN, b, _H), f32),
                   jax.ShapeDtypeStruct((b, _MAXSN), f32)],
        scratch_shapes=[
            pltpu.VMEM((bb1, _MAXSN, _E), f32),
            pltpu.VMEM((bb1, _MAXSN, 4 * _HE), f32),
            pltpu.VMEM((bb1, _MAXSN, 4 * _HE), f32),
            pltpu.VMEM((_MAXSN, bb1, 4 * _HE), f32),
            pltpu.VMEM((_MAXSN, bb1, 4 * _HE), f32),
            pltpu.VMEM((_MAXSN, bb1, _HE), f32),
            pltpu.VMEM((_MAXSN, bb1, _HE), f32),
            pltpu.VMEM((_MAXSN, bb1, _HE), f32),
            pltpu.VMEM((_MAXSN, bb1, _HE), f32),
        ],
        compiler_params=_cparams(),
        name="eyet_encode",
    )(sn_bert_emb, word_ids_sn, w0f, w0b, wih_enc, whh_enc, benc)

    tok_p = jnp.take(params['word_emb'], sp_emd, axis=0)
    posg_p = jnp.take(params['pos_emb'], sp_pos, axis=0)
    wid_sp = jnp.concatenate(
        [word_ids_sp[:, :-1],
         jnp.full((b, 1), _MAXSP - 1, word_ids_sp.dtype)], axis=1)

    dec = params['dec']
    w0d = dec[0]['Wih'].T
    b0d = (dec[0]['bih'] + dec[0]['bhh'])[None, :]
    lng = params['ln_g'][None, :]
    lnb = params['ln_b'][None, :]

    bb2 = min(16, b)
    xall = pl.pallas_call(
        _prep_kernel,
        grid=(b // bb2,),
        in_specs=[
            _blk((bb2, _MAXSP, _E)),
            _blk((bb2, _MAXSP, _E)),
            _blk((bb2, _MAXSP)),
            _full((1, _E)),
            _full((1, _E)),
            _full((_E, 4 * _H)),
            _full((1, 4 * _H)),
        ],
        out_specs=_blk((bb2, _MAXSP, 4 * _H)),
        out_shape=jax.ShapeDtypeStruct((b, _MAXSP, 4 * _H), f32),
        scratch_shapes=[pltpu.VMEM((bb2, _MAXSP, _E), f32)],
        compiler_params=_cparams(),
        name="eyet_prep",
    )(tok_p, posg_p, wid_sp, lng, lnb, w0d, b0d)

    xallT = jnp.swapaxes(xall, 0, 1)

    posi = sp_pos.astype(jnp.int32)
    iw = jnp.arange(_MAXSN, dtype=jnp.int32)[None, None, :]
    left = jnp.maximum(posi - 1, 0)[:, :, None]
    right = jnp.minimum(posi + 1, _MAXSN - 1)[:, :, None]
    bad = (maskf[:, None, :] == 0.0) | (iw < left) | (iw > right)
    negadd = jnp.where(bad, _NEG, 0.0).astype(f32)
    negaddT = jnp.swapaxes(negadd, 0, 1)

    wihd = jnp.stack([dec[l]['Wih'].T for l in range(1, 8)])
    whhd = jnp.stack([dec[l]['Whh'].T for l in range(8)])
    bd = jnp.stack([dec[l]['bih'] + dec[l]['bhh']
                    for l in range(1, 8)])
    bd = jnp.concatenate([bd, jnp.zeros((1, 4 * _H), f32)])

    wat = params['attn']['W'].T
    wam = wat[:, 0:_H]
    wal = wat[:, _H][None, :]
    ba = params['attn']['b'][None, 0:_H]
    bal = params['attn']['b'][None, _H:_H + 1]

    wd1t = params['d1']['W'].T
    wd1a = wd1t[0:_H, :]
    wd1l = wd1t[_H:_H + 1, :]
    wd1b = wd1t[_H + 1:2 * _H + 1, :]
    b1 = params['d1']['b'][None, :]
    wd2 = params['d2']['W'].T
    b2 = params['d2']['b'][None, :]
    wd3 = params['d3']['W'].T
    b3 = params['d3']['b'][None, :]
    wd4 = params['d4']['W'].T
    b4 = params['d4']['b'][None, :]
    wd5 = params['d5']['W'].T
    b5 = params['d5']['b'][None, :]
    nout = 2 * _MAXSN - 3

    bb3 = min(32, b)
    outT = pl.pallas_call(
        _dec_kernel,
        grid=(b // bb3,),
        in_specs=[
            _blk1((_MAXSP, bb3, 4 * _H)),
            _blk1((_MAXSN, bb3, _H)),
            _blk((bb3, _MAXSN)),
            _blk1((_MAXSP, bb3, _MAXSN)),
            _full((7, _H, 4 * _H)),
            _full((8, _H, 4 * _H)),
            _full((8, 4 * _H)),
            _full((_H, _H)),
            _full((1, _H)),
            _full((1, _H)),
            _full((1, 1)),
            _full((_H, 512)),
            _full((1, 512)),
            _full((_H, 512)),
            _full((1, 512)),
            _full((512, 256)),
            _full((1, 256)),
            _full((256, 256)),
            _full((1, 256)),
            _full((256, 256)),
            _full((1, 256)),
            _full((256, nout)),
            _full((1, nout)),
        ],
        out_specs=_blk1((_MAXSP, bb3, nout)),
        out_shape=jax.ShapeDtypeStruct((_MAXSP, b, nout), f32),
        compiler_params=_cparams(),
        name="eyet_decode",
    )(xallT, wencT, sn_word_len, negaddT,
      wihd, whhd, bd, wam, wal, ba, bal,
      wd1a, wd1l, wd1b, b1, wd2, b2, wd3, b3, wd4, b4, wd5, b5)

    return jnp.swapaxes(outT[0:_MAXSP - 1], 0, 1)


# speedup vs baseline: 1.4049x; 1.2264x over previous
import jax
import jax.numpy as jnp
from jax import lax
from jax.experimental import pallas as pl
from jax.experimental.pallas import tpu as pltpu

_S = 128
_MAXSN = 64
_MAXSP = 128
_E = 768
_H = 128
_HE = 64
_NEG = -1e9
_F32 = jnp.float32


def _sig(x):
    return jax.nn.sigmoid(x)


def _enc_kernel(emb_ref, wid_ref, w0f_ref, w0b_ref, wih_ref, whh_ref, benc_ref,
                wenc_ref, mask_ref,
                pooled_ref, xihf_ref, xihb_ref, xtf_ref, xtb_ref,
                s0f_ref, s0b_ref, s1f_ref, s1b_ref):
    bb = emb_ref.shape[0]

    def pool_body(b, _):
        wrow = wid_ref[pl.ds(b, 1), :]
        iw = lax.broadcasted_iota(jnp.int32, (_MAXSN, _S), 0)
        oh = (iw == wrow).astype(_F32)
        pooled_ref[b] = jnp.dot(oh, emb_ref[b],
                                preferred_element_type=_F32)
        return 0

    lax.fori_loop(0, bb, pool_body, 0)
    mask_ref[...] = (jnp.sum(pooled_ref[...], axis=2) != 0.0).astype(_F32)

    pin = pooled_ref[...].reshape(bb * _MAXSN, _E)
    xihf_ref[...] = (jnp.dot(pin, w0f_ref[...], preferred_element_type=_F32)
                     + benc_ref[0:1, :]).reshape(bb, _MAXSN, 4 * _HE)
    xihb_ref[...] = (jnp.dot(pin, w0b_ref[...], preferred_element_type=_F32)
                     + benc_ref[1:2, :]).reshape(bb, _MAXSN, 4 * _HE)

    def gates(g, h, c):
        i_ = g[:, 0:_HE]
        f_ = g[:, _HE:2 * _HE]
        g_ = g[:, 2 * _HE:3 * _HE]
        o_ = g[:, 3 * _HE:4 * _HE]
        c2 = _sig(f_) * c + _sig(i_) * jnp.tanh(g_)
        h2 = _sig(o_) * jnp.tanh(c2)
        return h2, c2

    def lstm_dir0(xih_ref, whh_idx, out_ref, reverse):
        whh = whh_ref[whh_idx]

        def chunk(c, carry):
            base = pl.multiple_of(
                (_MAXSN - 8) - 8 * c if reverse else 8 * c, 8)
            xch = xih_ref[:, pl.ds(base, 8), :]
            h, cc = carry
            order = range(7, -1, -1) if reverse else range(8)
            for j in order:
                g = xch[:, j, :] + jnp.dot(h, whh,
                                           preferred_element_type=_F32)
                h, cc = gates(g, h, cc)
                out_ref[base + j] = h
            return (h, cc)

        init = (jnp.zeros((bb, _HE), _F32), jnp.zeros((bb, _HE), _F32))
        lax.fori_loop(0, _MAXSN // 8, chunk, init)

    def lstm_bidirT(l, out_f, out_b):
        whf = whh_ref[2 * l]
        whb = whh_ref[2 * l + 1]

        def step(i, carry):
            tr = (_MAXSN - 1) - i
            hf, cf, hb, cb = carry
            gf = xtf_ref[i] + jnp.dot(hf, whf, preferred_element_type=_F32)
            gb = xtb_ref[tr] + jnp.dot(hb, whb, preferred_element_type=_F32)
            hf, cf = gates(gf, hf, cf)
            hb, cb = gates(gb, hb, cb)
            out_f[i] = hf
            out_b[tr] = hb
            return (hf, cf, hb, cb)

        z = jnp.zeros((bb, _HE), _F32)
        lax.fori_loop(0, _MAXSN, step, (z, z, z, z))

    pairs = [(s0f_ref, s0b_ref), (s1f_ref, s1b_ref)]

    for l in range(8):
        out_f, out_b = pairs[l % 2]
        if l == 0:
            lstm_dir0(xihf_ref, 0, out_f, False)
            lstm_dir0(xihb_ref, 1, out_b, True)
        else:
            in_f, in_b = pairs[(l - 1) % 2]
            xf = in_f[...].reshape(_MAXSN * bb, _HE)
            xb = in_b[...].reshape(_MAXSN * bb, _HE)
            for di, xih_ref in ((0, xtf_ref), (1, xtb_ref)):
                w = wih_ref[2 * (l - 1) + di]
                xih_ref[...] = (jnp.dot(xf, w[0:_HE, :],
                                        preferred_element_type=_F32)
                                + jnp.dot(xb, w[_HE:2 * _HE, :],
                                          preferred_element_type=_F32)
                                + benc_ref[2 * l + di:2 * l + di + 1, :]
                                ).reshape(_MAXSN, bb, 4 * _HE)
            lstm_bidirT(l, out_f, out_b)
            if l >= 2:
                out_f[...] = out_f[...] + in_f[...]
                out_b[...] = out_b[...] + in_b[...]

    fin_f, fin_b = pairs[7 % 2]
    wenc_ref[:, :, 0:_HE] = fin_f[...]
    wenc_ref[:, :, _HE:2 * _HE] = fin_b[...]


def _prep_kernel(tok_ref, posg_ref, wid_ref, lng_ref, lnb_ref, w0_ref, b0_ref,
                 out_ref, xn_ref):
    bb = tok_ref.shape[0]

    def body(b, _):
        wrow = wid_ref[pl.ds(b, 1), :]
        iw = lax.broadcasted_iota(jnp.int32, (_MAXSP, _MAXSP), 0)
        oh = (iw == wrow).astype(_F32)
        x = jnp.dot(oh, tok_ref[b], preferred_element_type=_F32) \
            + posg_ref[b]
        m = jnp.mean(x, axis=-1, keepdims=True)
        xc = x - m
        v = jnp.mean(xc * xc, axis=-1, keepdims=True)
        xn_ref[b] = xc * lax.rsqrt(v + 1e-12) * lng_ref[...] + lnb_ref[...]
        return 0

    lax.fori_loop(0, bb, body, 0)
    out_ref[...] = (jnp.dot(xn_ref[...].reshape(bb * _MAXSP, _E), w0_ref[...],
                            preferred_element_type=_F32)
                    + b0_ref[...]).reshape(bb, _MAXSP, 4 * _H)


def _dec_kernel(xall_ref, wencT_ref, snlen_ref, negadd_ref,
                wihd_ref, whhd_ref, bd_ref,
                wam_ref, wal_ref, ba_ref, bal_ref,
                wd1a_ref, wd1l_ref, wd1b_ref, b1_ref,
                wd2_ref, b2_ref, wd3_ref, b3_ref, wd4_ref, b4_ref,
                wd5_ref, b5_ref,
                out_ref):
    bb = xall_ref.shape[1]
    snlen = snlen_ref[...]

    def cell(g, c):
        i_ = g[:, 0:_H]
        f_ = g[:, _H:2 * _H]
        g_ = g[:, 2 * _H:3 * _H]
        o_ = g[:, 3 * _H:4 * _H]
        c2 = _sig(f_) * c + _sig(i_) * jnp.tanh(g_)
        h2 = _sig(o_) * jnp.tanh(c2)
        return h2, c2

    def step(t, carry):
        hs = list(carry[0:8])
        cs = list(carry[8:16])
        xt = xall_ref[t]
        g0 = xt + jnp.dot(hs[0], whhd_ref[0], preferred_element_type=_F32)
        hs[0], cs[0] = cell(g0, cs[0])
        cur = None
        for l in range(1, 8):
            src = hs[0] if l == 1 else cur
            gl = (jnp.dot(src, wihd_ref[l - 1], preferred_element_type=_F32)
                  + jnp.dot(hs[l], whhd_ref[l], preferred_element_type=_F32)
                  + bd_ref[l - 1:l, :])
            hs[l], cs[l] = cell(gl, cs[l])
            cur = hs[l] if l == 1 else hs[l] + cur
        in8 = cur
        a128 = jnp.dot(in8, wam_ref[...],
                       preferred_element_type=_F32) + ba_ref[...]
        alast = jnp.sum(in8 * wal_ref[...], axis=-1,
                        keepdims=True) + bal_ref[...]
        cols = [jnp.sum(a128 * wencT_ref[w], axis=-1, keepdims=True)
                for w in range(_MAXSN)]
        prod = jnp.concatenate(cols, axis=1) \
            + alast * snlen + negadd_ref[t]
        mx = jnp.max(prod, axis=-1, keepdims=True)
        ex = jnp.exp(prod - mx)
        wgt = ex / jnp.sum(ex, axis=-1, keepdims=True)
        ctx = wgt[:, 0:1] * wencT_ref[0][0]
        for w in range(1, _MAXSN):
            ctx = ctx + wgt[:, w:w + 1] * wencT_ref[w]
        ctxl = jnp.sum(wgt * snlen, axis=-1, keepdims=True)
        z = jnp.maximum(
            jnp.dot(ctx, wd1a_ref[...], preferred_element_type=_F32)
            + ctxl * wd1l_ref[...]
            + jnp.dot(in8, wd1b_ref[...], preferred_element_type=_F32)
            + b1_ref[...], 0.0)
        z = jnp.maximum(jnp.dot(z, wd2_ref[...],
                                preferred_element_type=_F32) + b2_ref[...],
                        0.0)
        z = jnp.maximum(jnp.dot(z, wd3_ref[...],
                                preferred_element_type=_F32) + b3_ref[...],
                        0.0)
        z = jnp.maximum(jnp.dot(z, wd4_ref[...],
                                preferred_element_type=_F32) + b4_ref[...],
                        0.0)
        o = jnp.dot(z, wd5_ref[...],
                    preferred_element_type=_F32) + b5_ref[...]
        out_ref[t] = o
        return tuple(hs) + tuple(cs)

    zeros = jnp.zeros((bb, _H), _F32)
    lax.fori_loop(0, _MAXSP, step, tuple(zeros for _ in range(16)))


def _full(shape):
    nd = len(shape)
    return pl.BlockSpec(shape, lambda i: (0,) * nd)


def _blk(shape):
    nd = len(shape)
    return pl.BlockSpec(shape, lambda i, _nd=nd: (i,) + (0,) * (_nd - 1))


def _blk1(shape):
    nd = len(shape)
    return pl.BlockSpec(shape, lambda i, _nd=nd: (0, i) + (0,) * (_nd - 2))


def _cparams(vmem_mb=56):
    return pltpu.CompilerParams(dimension_semantics=("arbitrary",),
                                vmem_limit_bytes=vmem_mb * 1024 * 1024)


@jax.jit
def kernel(sn_bert_emb, sn_word_len, params, word_ids_sn, sp_emd, sp_pos,
           word_ids_sp):
    b = sn_bert_emb.shape[0]
    f32 = _F32

    enc = params['enc']
    w0f = enc[0]['f']['Wih'].T
    w0b = enc[0]['b']['Wih'].T
    wih_enc = jnp.stack([enc[l][d]['Wih'].T
                         for l in range(1, 8) for d in ('f', 'b')])
    whh_enc = jnp.stack([enc[l][d]['Whh'].T
                         for l in range(8) for d in ('f', 'b')])
    benc = jnp.stack([enc[l][d]['bih'] + enc[l][d]['bhh']
                      for l in range(8) for d in ('f', 'b')])

    bb1 = min(32, b)
    wencT, maskf = pl.pallas_call(
        _enc_kernel,
        grid=(b // bb1,),
        in_specs=[
            _blk((bb1, _S, _E)),
            _blk((bb1, _S)),
            _full((_E, 4 * _HE)),
            _full((_E, 4 * _HE)),
            _full((14, 2 * _HE, 4 * _HE)),
            _full((16, _HE, 4 * _HE)),
            _full((16, 4 * _HE)),
        ],
        out_specs=[_blk1((_MAXSN, bb1, _H)), _blk((bb1, _MAXSN))],
        out_shape=[jax.ShapeDtypeStruct((_MAXSN, b, _H), f32),
                   jax.ShapeDtypeStruct((b, _MAXSN), f32)],
        scratch_shapes=[
            pltpu.VMEM((bb1, _MAXSN, _E), f32),
            pltpu.VMEM((bb1, _MAXSN, 4 * _HE), f32),
            pltpu.VMEM((bb1, _MAXSN, 4 * _HE), f32),
            pltpu.VMEM((_MAXSN, bb1, 4 * _HE), f32),
            pltpu.VMEM((_MAXSN, bb1, 4 * _HE), f32),
            pltpu.VMEM((_MAXSN, bb1, _HE), f32),
            pltpu.VMEM((_MAXSN, bb1, _HE), f32),
            pltpu.VMEM((_MAXSN, bb1, _HE), f32),
            pltpu.VMEM((_MAXSN, bb1, _HE), f32),
        ],
        compiler_params=_cparams(),
        name="eyet_encode",
    )(sn_bert_emb, word_ids_sn, w0f, w0b, wih_enc, whh_enc, benc)

    tok_p = jnp.take(params['word_emb'], sp_emd, axis=0)
    posg_p = jnp.take(params['pos_emb'], sp_pos, axis=0)
    wid_sp = jnp.concatenate(
        [word_ids_sp[:, :-1],
         jnp.full((b, 1), _MAXSP - 1, word_ids_sp.dtype)], axis=1)

    dec = params['dec']
    w0d = dec[0]['Wih'].T
    b0d = (dec[0]['bih'] + dec[0]['bhh'])[None, :]
    lng = params['ln_g'][None, :]
    lnb = params['ln_b'][None, :]

    bb2 = min(16, b)
    xall = pl.pallas_call(
        _prep_kernel,
        grid=(b // bb2,),
        in_specs=[
            _blk((bb2, _MAXSP, _E)),
            _blk((bb2, _MAXSP, _E)),
            _blk((bb2, _MAXSP)),
            _full((1, _E)),
            _full((1, _E)),
            _full((_E, 4 * _H)),
            _full((1, 4 * _H)),
        ],
        out_specs=_blk((bb2, _MAXSP, 4 * _H)),
        out_shape=jax.ShapeDtypeStruct((b, _MAXSP, 4 * _H), f32),
        scratch_shapes=[pltpu.VMEM((bb2, _MAXSP, _E), f32)],
        compiler_params=_cparams(),
        name="eyet_prep",
    )(tok_p, posg_p, wid_sp, lng, lnb, w0d, b0d)

    xallT = jnp.swapaxes(xall, 0, 1)

    posi = sp_pos.astype(jnp.int32)
    iw = jnp.arange(_MAXSN, dtype=jnp.int32)[None, None, :]
    left = jnp.maximum(posi - 1, 0)[:, :, None]
    right = jnp.minimum(posi + 1, _MAXSN - 1)[:, :, None]
    bad = (maskf[:, None, :] == 0.0) | (iw < left) | (iw > right)
    negadd = jnp.where(bad, _NEG, 0.0).astype(f32)
    negaddT = jnp.swapaxes(negadd, 0, 1)

    wihd = jnp.stack([dec[l]['Wih'].T for l in range(1, 8)])
    whhd = jnp.stack([dec[l]['Whh'].T for l in range(8)])
    bd = jnp.stack([dec[l]['bih'] + dec[l]['bhh']
                    for l in range(1, 8)])
    bd = jnp.concatenate([bd, jnp.zeros((1, 4 * _H), f32)])

    wat = params['attn']['W'].T
    wam = wat[:, 0:_H]
    wal = wat[:, _H][None, :]
    ba = params['attn']['b'][None, 0:_H]
    bal = params['attn']['b'][None, _H:_H + 1]

    wd1t = params['d1']['W'].T
    wd1a = wd1t[0:_H, :]
    wd1l = wd1t[_H:_H + 1, :]
    wd1b = wd1t[_H + 1:2 * _H + 1, :]
    b1 = params['d1']['b'][None, :]
    wd2 = params['d2']['W'].T
    b2 = params['d2']['b'][None, :]
    wd3 = params['d3']['W'].T
    b3 = params['d3']['b'][None, :]
    wd4 = params['d4']['W'].T
    b4 = params['d4']['b'][None, :]
    wd5 = params['d5']['W'].T
    b5 = params['d5']['b'][None, :]
    nout = 2 * _MAXSN - 3

    bb3 = min(32, b)
    outT = pl.pallas_call(
        _dec_kernel,
        grid=(b // bb3,),
        in_specs=[
            _blk1((_MAXSP, bb3, 4 * _H)),
            _blk1((_MAXSN, bb3, _H)),
            _blk((bb3, _MAXSN)),
            _blk1((_MAXSP, bb3, _MAXSN)),
            _full((7, _H, 4 * _H)),
            _full((8, _H, 4 * _H)),
            _full((8, 4 * _H)),
            _full((_H, _H)),
            _full((1, _H)),
            _full((1, _H)),
            _full((1, 1)),
            _full((_H, 512)),
            _full((1, 512)),
            _full((_H, 512)),
            _full((1, 512)),
            _full((512, 256)),
            _full((1, 256)),
            _full((256, 256)),
            _full((1, 256)),
            _full((256, 256)),
            _full((1, 256)),
            _full((256, nout)),
            _full((1, nout)),
        ],
        out_specs=_blk1((_MAXSP, bb3, nout)),
        out_shape=jax.ShapeDtypeStruct((_MAXSP, b, nout), f32),
        compiler_params=_cparams(),
        name="eyet_decode",
    )(xallT, wencT, sn_word_len, negaddT,
      wihd, whhd, bd, wam, wal, ba, bal,
      wd1a, wd1l, wd1b, b1, wd2, b2, wd3, b3, wd4, b4, wd5, b5)

    return jnp.swapaxes(outT[0:_MAXSP - 1], 0, 1)


# speedup vs baseline: 1.4398x; 1.0248x over previous
import jax
import jax.numpy as jnp
from jax import lax
from jax.experimental import pallas as pl
from jax.experimental.pallas import tpu as pltpu

_S = 128
_MAXSN = 64
_MAXSP = 128
_E = 768
_H = 128
_HE = 64
_NEG = -1e9
_F32 = jnp.float32


def _sig(x):
    return jax.nn.sigmoid(x)


def _enc_kernel(emb_ref, wid_ref, w0f_ref, w0b_ref, wih_ref, whh_ref, benc_ref,
                wenc_ref, mask_ref,
                pooled_ref, xihf_ref, xihb_ref, xtf_ref, xtb_ref,
                s0f_ref, s0b_ref, s1f_ref, s1b_ref):
    bb = emb_ref.shape[0]

    def pool_body(b, _):
        wrow = wid_ref[pl.ds(b, 1), :]
        iw = lax.broadcasted_iota(jnp.int32, (_MAXSN, _S), 0)
        oh = (iw == wrow).astype(_F32)
        pooled_ref[b] = jnp.dot(oh, emb_ref[b],
                                preferred_element_type=_F32)
        return 0

    lax.fori_loop(0, bb, pool_body, 0)
    mask_ref[...] = (jnp.sum(pooled_ref[...], axis=2) != 0.0).astype(_F32)

    pin = pooled_ref[...].reshape(bb * _MAXSN, _E)
    xihf_ref[...] = (jnp.dot(pin, w0f_ref[...], preferred_element_type=_F32)
                     + benc_ref[0:1, :]).reshape(bb, _MAXSN, 4 * _HE)
    xihb_ref[...] = (jnp.dot(pin, w0b_ref[...], preferred_element_type=_F32)
                     + benc_ref[1:2, :]).reshape(bb, _MAXSN, 4 * _HE)

    def gates(g, h, c):
        i_ = g[:, 0:_HE]
        f_ = g[:, _HE:2 * _HE]
        g_ = g[:, 2 * _HE:3 * _HE]
        o_ = g[:, 3 * _HE:4 * _HE]
        c2 = _sig(f_) * c + _sig(i_) * jnp.tanh(g_)
        h2 = _sig(o_) * jnp.tanh(c2)
        return h2, c2

    def lstm_bidir0(out_f, out_b):
        whf = whh_ref[0]
        whb = whh_ref[1]

        def chunk(c, carry):
            basef = pl.multiple_of(8 * c, 8)
            baseb = pl.multiple_of((_MAXSN - 8) - 8 * c, 8)
            xchf = xihf_ref[:, pl.ds(basef, 8), :]
            xchb = xihb_ref[:, pl.ds(baseb, 8), :]
            hf, cf, hb, cb = carry
            for j in range(8):
                gf = xchf[:, j, :] + jnp.dot(hf, whf,
                                             preferred_element_type=_F32)
                gb = xchb[:, 7 - j, :] + jnp.dot(hb, whb,
                                                 preferred_element_type=_F32)
                hf, cf = gates(gf, hf, cf)
                hb, cb = gates(gb, hb, cb)
                out_f[basef + j] = hf
                out_b[baseb + 7 - j] = hb
            return (hf, cf, hb, cb)

        z = jnp.zeros((bb, _HE), _F32)
        lax.fori_loop(0, _MAXSN // 8, chunk, (z, z, z, z))

    def lstm_bidirT(l, out_f, out_b):
        whf = whh_ref[2 * l]
        whb = whh_ref[2 * l + 1]

        def step(i, carry):
            tr = (_MAXSN - 1) - i
            hf, cf, hb, cb = carry
            gf = xtf_ref[i] + jnp.dot(hf, whf, preferred_element_type=_F32)
            gb = xtb_ref[tr] + jnp.dot(hb, whb, preferred_element_type=_F32)
            hf, cf = gates(gf, hf, cf)
            hb, cb = gates(gb, hb, cb)
            out_f[i] = hf
            out_b[tr] = hb
            return (hf, cf, hb, cb)

        z = jnp.zeros((bb, _HE), _F32)
        lax.fori_loop(0, _MAXSN, step, (z, z, z, z))

    pairs = [(s0f_ref, s0b_ref), (s1f_ref, s1b_ref)]

    for l in range(8):
        out_f, out_b = pairs[l % 2]
        if l == 0:
            lstm_bidir0(out_f, out_b)
        else:
            in_f, in_b = pairs[(l - 1) % 2]
            xf = in_f[...].reshape(_MAXSN * bb, _HE)
            xb = in_b[...].reshape(_MAXSN * bb, _HE)
            for di, xih_ref in ((0, xtf_ref), (1, xtb_ref)):
                w = wih_ref[2 * (l - 1) + di]
                xih_ref[...] = (jnp.dot(xf, w[0:_HE, :],
                                        preferred_element_type=_F32)
                                + jnp.dot(xb, w[_HE:2 * _HE, :],
                                          preferred_element_type=_F32)
                                + benc_ref[2 * l + di:2 * l + di + 1, :]
                                ).reshape(_MAXSN, bb, 4 * _HE)
            lstm_bidirT(l, out_f, out_b)
            if l >= 2:
                out_f[...] = out_f[...] + in_f[...]
                out_b[...] = out_b[...] + in_b[...]

    fin_f, fin_b = pairs[7 % 2]
    wenc_ref[:, :, 0:_HE] = fin_f[...]
    wenc_ref[:, :, _HE:2 * _HE] = fin_b[...]


def _prep_kernel(tok_ref, posg_ref, wid_ref, lng_ref, lnb_ref, w0_ref, b0_ref,
                 out_ref, xn_ref):
    bb = tok_ref.shape[0]

    def body(b, _):
        wrow = wid_ref[pl.ds(b, 1), :]
        iw = lax.broadcasted_iota(jnp.int32, (_MAXSP, _MAXSP), 0)
        oh = (iw == wrow).astype(_F32)
        x = jnp.dot(oh, tok_ref[b], preferred_element_type=_F32) \
            + posg_ref[b]
        m = jnp.mean(x, axis=-1, keepdims=True)
        xc = x - m
        v = jnp.mean(xc * xc, axis=-1, keepdims=True)
        xn_ref[b] = xc * lax.rsqrt(v + 1e-12) * lng_ref[...] + lnb_ref[...]
        return 0

    lax.fori_loop(0, bb, body, 0)
    out_ref[...] = (jnp.dot(xn_ref[...].reshape(bb * _MAXSP, _E), w0_ref[...],
                            preferred_element_type=_F32)
                    + b0_ref[...]).reshape(bb, _MAXSP, 4 * _H)


def _dec_kernel(xall_ref, wencT_ref, snlen_ref, negadd_ref,
                wihd_ref, whhd_ref, bd_ref,
                wam_ref, wal_ref, ba_ref, bal_ref,
                wd1a_ref, wd1l_ref, wd1b_ref, b1_ref,
                wd2_ref, b2_ref, wd3_ref, b3_ref, wd4_ref, b4_ref,
                wd5_ref, b5_ref,
                out_ref):
    bb = xall_ref.shape[1]
    snlen = snlen_ref[...]

    def cell(g, c):
        i_ = g[:, 0:_H]
        f_ = g[:, _H:2 * _H]
        g_ = g[:, 2 * _H:3 * _H]
        o_ = g[:, 3 * _H:4 * _H]
        c2 = _sig(f_) * c + _sig(i_) * jnp.tanh(g_)
        h2 = _sig(o_) * jnp.tanh(c2)
        return h2, c2

    def step(t, carry):
        hs = list(carry[0:8])
        cs = list(carry[8:16])
        xt = xall_ref[t]
        g0 = xt + jnp.dot(hs[0], whhd_ref[0], preferred_element_type=_F32)
        hs[0], cs[0] = cell(g0, cs[0])
        cur = None
        for l in range(1, 8):
            src = hs[0] if l == 1 else cur
            gl = (jnp.dot(src, wihd_ref[l - 1], preferred_element_type=_F32)
                  + jnp.dot(hs[l], whhd_ref[l], preferred_element_type=_F32)
                  + bd_ref[l - 1:l, :])
            hs[l], cs[l] = cell(gl, cs[l])
            cur = hs[l] if l == 1 else hs[l] + cur
        in8 = cur
        a128 = jnp.dot(in8, wam_ref[...],
                       preferred_element_type=_F32) + ba_ref[...]
        alast = jnp.sum(in8 * wal_ref[...], axis=-1,
                        keepdims=True) + bal_ref[...]
        cols = [jnp.sum(a128 * wencT_ref[w], axis=-1, keepdims=True)
                for w in range(_MAXSN)]
        prod = jnp.concatenate(cols, axis=1) \
            + alast * snlen + negadd_ref[t]
        mx = jnp.max(prod, axis=-1, keepdims=True)
        ex = jnp.exp(prod - mx)
        wgt = ex / jnp.sum(ex, axis=-1, keepdims=True)
        ctx = wgt[:, 0:1] * wencT_ref[0][0]
        for w in range(1, _MAXSN):
            ctx = ctx + wgt[:, w:w + 1] * wencT_ref[w]
        ctxl = jnp.sum(wgt * snlen, axis=-1, keepdims=True)
        z = jnp.maximum(
            jnp.dot(ctx, wd1a_ref[...], preferred_element_type=_F32)
            + ctxl * wd1l_ref[...]
            + jnp.dot(in8, wd1b_ref[...], preferred_element_type=_F32)
            + b1_ref[...], 0.0)
        z = jnp.maximum(jnp.dot(z, wd2_ref[...],
                                preferred_element_type=_F32) + b2_ref[...],
                        0.0)
        z = jnp.maximum(jnp.dot(z, wd3_ref[...],
                                preferred_element_type=_F32) + b3_ref[...],
                        0.0)
        z = jnp.maximum(jnp.dot(z, wd4_ref[...],
                                preferred_element_type=_F32) + b4_ref[...],
                        0.0)
        o = jnp.dot(z, wd5_ref[...],
                    preferred_element_type=_F32) + b5_ref[...]
        out_ref[t] = o
        return tuple(hs) + tuple(cs)

    zeros = jnp.zeros((bb, _H), _F32)
    lax.fori_loop(0, _MAXSP, step, tuple(zeros for _ in range(16)))


def _full(shape):
    nd = len(shape)
    return pl.BlockSpec(shape, lambda i: (0,) * nd)


def _blk(shape):
    nd = len(shape)
    return pl.BlockSpec(shape, lambda i, _nd=nd: (i,) + (0,) * (_nd - 1))


def _blk1(shape):
    nd = len(shape)
    return pl.BlockSpec(shape, lambda i, _nd=nd: (0, i) + (0,) * (_nd - 2))


def _cparams(vmem_mb=56):
    return pltpu.CompilerParams(dimension_semantics=("arbitrary",),
                                vmem_limit_bytes=vmem_mb * 1024 * 1024)


@jax.jit
def kernel(sn_bert_emb, sn_word_len, params, word_ids_sn, sp_emd, sp_pos,
           word_ids_sp):
    b = sn_bert_emb.shape[0]
    f32 = _F32

    enc = params['enc']
    w0f = enc[0]['f']['Wih'].T
    w0b = enc[0]['b']['Wih'].T
    wih_enc = jnp.stack([enc[l][d]['Wih'].T
                         for l in range(1, 8) for d in ('f', 'b')])
    whh_enc = jnp.stack([enc[l][d]['Whh'].T
                         for l in range(8) for d in ('f', 'b')])
    benc = jnp.stack([enc[l][d]['bih'] + enc[l][d]['bhh']
                      for l in range(8) for d in ('f', 'b')])

    bb1 = min(32, b)
    wencT, maskf = pl.pallas_call(
        _enc_kernel,
        grid=(b // bb1,),
        in_specs=[
            _blk((bb1, _S, _E)),
            _blk((bb1, _S)),
            _full((_E, 4 * _HE)),
            _full((_E, 4 * _HE)),
            _full((14, 2 * _HE, 4 * _HE)),
            _full((16, _HE, 4 * _HE)),
            _full((16, 4 * _HE)),
        ],
        out_specs=[_blk1((_MAXSN, bb1, _H)), _blk((bb1, _MAXSN))],
        out_shape=[jax.ShapeDtypeStruct((_MAXSN, b, _H), f32),
                   jax.ShapeDtypeStruct((b, _MAXSN), f32)],
        scratch_shapes=[
            pltpu.VMEM((bb1, _MAXSN, _E), f32),
            pltpu.VMEM((bb1, _MAXSN, 4 * _HE), f32),
            pltpu.VMEM((bb1, _MAXSN, 4 * _HE), f32),
            pltpu.VMEM((_MAXSN, bb1, 4 * _HE), f32),
            pltpu.VMEM((_MAXSN, bb1, 4 * _HE), f32),
            pltpu.VMEM((_MAXSN, bb1, _HE), f32),
            pltpu.VMEM((_MAXSN, bb1, _HE), f32),
            pltpu.VMEM((_MAXSN, bb1, _HE), f32),
            pltpu.VMEM((_MAXSN, bb1, _HE), f32),
        ],
        compiler_params=_cparams(),
        name="eyet_encode",
    )(sn_bert_emb, word_ids_sn, w0f, w0b, wih_enc, whh_enc, benc)

    tok_p = jnp.take(params['word_emb'], sp_emd, axis=0)
    posg_p = jnp.take(params['pos_emb'], sp_pos, axis=0)
    wid_sp = jnp.concatenate(
        [word_ids_sp[:, :-1],
         jnp.full((b, 1), _MAXSP - 1, word_ids_sp.dtype)], axis=1)

    dec = params['dec']
    w0d = dec[0]['Wih'].T
    b0d = (dec[0]['bih'] + dec[0]['bhh'])[None, :]
    lng = params['ln_g'][None, :]
    lnb = params['ln_b'][None, :]

    bb2 = min(16, b)
    xall = pl.pallas_call(
        _prep_kernel,
        grid=(b // bb2,),
        in_specs=[
            _blk((bb2, _MAXSP, _E)),
            _blk((bb2, _MAXSP, _E)),
            _blk((bb2, _MAXSP)),
            _full((1, _E)),
            _full((1, _E)),
            _full((_E, 4 * _H)),
            _full((1, 4 * _H)),
        ],
        out_specs=_blk((bb2, _MAXSP, 4 * _H)),
        out_shape=jax.ShapeDtypeStruct((b, _MAXSP, 4 * _H), f32),
        scratch_shapes=[pltpu.VMEM((bb2, _MAXSP, _E), f32)],
        compiler_params=_cparams(),
        name="eyet_prep",
    )(tok_p, posg_p, wid_sp, lng, lnb, w0d, b0d)

    xallT = jnp.swapaxes(xall, 0, 1)

    posi = sp_pos.astype(jnp.int32)
    iw = jnp.arange(_MAXSN, dtype=jnp.int32)[None, None, :]
    left = jnp.maximum(posi - 1, 0)[:, :, None]
    right = jnp.minimum(posi + 1, _MAXSN - 1)[:, :, None]
    bad = (maskf[:, None, :] == 0.0) | (iw < left) | (iw > right)
    negadd = jnp.where(bad, _NEG, 0.0).astype(f32)
    negaddT = jnp.swapaxes(negadd, 0, 1)

    wihd = jnp.stack([dec[l]['Wih'].T for l in range(1, 8)])
    whhd = jnp.stack([dec[l]['Whh'].T for l in range(8)])
    bd = jnp.stack([dec[l]['bih'] + dec[l]['bhh']
                    for l in range(1, 8)])
    bd = jnp.concatenate([bd, jnp.zeros((1, 4 * _H), f32)])

    wat = params['attn']['W'].T
    wam = wat[:, 0:_H]
    wal = wat[:, _H][None, :]
    ba = params['attn']['b'][None, 0:_H]
    bal = params['attn']['b'][None, _H:_H + 1]

    wd1t = params['d1']['W'].T
    wd1a = wd1t[0:_H, :]
    wd1l = wd1t[_H:_H + 1, :]
    wd1b = wd1t[_H + 1:2 * _H + 1, :]
    b1 = params['d1']['b'][None, :]
    wd2 = params['d2']['W'].T
    b2 = params['d2']['b'][None, :]
    wd3 = params['d3']['W'].T
    b3 = params['d3']['b'][None, :]
    wd4 = params['d4']['W'].T
    b4 = params['d4']['b'][None, :]
    wd5 = params['d5']['W'].T
    b5 = params['d5']['b'][None, :]
    nout = 2 * _MAXSN - 3

    bb3 = min(32, b)
    outT = pl.pallas_call(
        _dec_kernel,
        grid=(b // bb3,),
        in_specs=[
            _blk1((_MAXSP, bb3, 4 * _H)),
            _blk1((_MAXSN, bb3, _H)),
            _blk((bb3, _MAXSN)),
            _blk1((_MAXSP, bb3, _MAXSN)),
            _full((7, _H, 4 * _H)),
            _full((8, _H, 4 * _H)),
            _full((8, 4 * _H)),
            _full((_H, _H)),
            _full((1, _H)),
            _full((1, _H)),
            _full((1, 1)),
            _full((_H, 512)),
            _full((1, 512)),
            _full((_H, 512)),
            _full((1, 512)),
            _full((512, 256)),
            _full((1, 256)),
            _full((256, 256)),
            _full((1, 256)),
            _full((256, 256)),
            _full((1, 256)),
            _full((256, nout)),
            _full((1, nout)),
        ],
        out_specs=_blk1((_MAXSP, bb3, nout)),
        out_shape=jax.ShapeDtypeStruct((_MAXSP, b, nout), f32),
        compiler_params=_cparams(),
        name="eyet_decode",
    )(xallT, wencT, sn_word_len, negaddT,
      wihd, whhd, bd, wam, wal, ba, bal,
      wd1a, wd1l, wd1b, b1, wd2, b2, wd3, b3, wd4, b4, wd5, b5)

    return jnp.swapaxes(outT[0:_MAXSP - 1], 0, 1)
